# baseline (speedup 1.0000x reference)
import functools

import jax
import jax.numpy as jnp
from jax import lax
from jax.experimental import pallas as pl
from jax.experimental.pallas import tpu as pltpu
from jax.experimental.pallas import tpu_sc as plsc

N = 10000
E = 320000
D = 128
G = 256

NC = 2
NS = 16
NW = NC * NS

K = 128
CH = 80
EPW = CH * K
EPAD = NW * EPW
RPT = 624
LAST = N - 15 * RPT

BN = 2000
NB = N // BN


def _mm_t(a, b):
    return lax.dot_general(a, b, (((1,), (1,)), ((), ())),
                           precision=lax.Precision.HIGHEST,
                           preferred_element_type=jnp.float32)


def _sc_edge_body(hr_hbm, src_hbm, dst_hbm, out_hbm,
                  src_v, d0, d1, r0, r1, acc,
                  gsa, gsb, ssa, ssb):
    c = lax.axis_index("c")
    s = lax.axis_index("s")
    wid = c * NS + s
    base = wid * EPW

    pltpu.sync_copy(src_hbm.at[wid], src_v)

    def zrow(i, carry):
        def zcol(j, carry2):
            r0[i, pl.ds(j * 16, 16)] = jnp.zeros((16,), jnp.float32)
            return carry2
        return lax.fori_loop(0, D // 16, zcol, carry)
    lax.fori_loop(0, K, zrow, 0)

    def zslab(i, carry):
        pltpu.sync_copy(r0, acc.at[pl.ds(s * RPT + i * K, K)])
        return carry
    lax.fori_loop(0, RPT // K, zslab, 0)

    @pl.when(s < NS - 1)
    def _zero_rem():
        pltpu.sync_copy(r0.at[pl.ds(0, RPT - (RPT // K) * K)],
                        acc.at[pl.ds(s * RPT + (RPT // K) * K,
                                     RPT - (RPT // K) * K)])

    @pl.when(s == NS - 1)
    def _zero_tail():
        pltpu.sync_copy(r0, acc.at[pl.ds(15 * RPT + (RPT // K) * K, K)])
    plsc.subcore_barrier()

    def gath(i, buf, dbuf, sem):
        pltpu.async_copy(hr_hbm.at[src_v.at[i]], buf, sem)
        pltpu.async_copy(dst_hbm.at[pl.ds(base + i * K, K)], dbuf, sem)

    def gwait(i, buf, dbuf, sem):
        pltpu.make_async_copy(hr_hbm.at[src_v.at[i]], buf, sem).wait()
        pltpu.make_async_copy(dst_hbm.at[pl.ds(base + i * K, K)], dbuf,
                              sem).wait()

    def scat(buf, dbuf, sem):
        pltpu.async_copy(buf, acc.at[dbuf], sem, add=True)

    def swait(buf, dbuf, sem):
        pltpu.make_async_copy(buf, acc.at[dbuf], sem).wait()

    gath(0, r0, d0, gsa)

    def body(t, carry):
        c0 = 2 * t
        gwait(c0, r0, d0, gsa)

        @pl.when(t > 0)
        def _drain_prev():
            swait(r1, d1, ssb)
        gath(c0 + 1, r1, d1, gsb)
        scat(r0, d0, ssa)
        gwait(c0 + 1, r1, d1, gsb)
        swait(r0, d0, ssa)

        @pl.when(t < CH // 2 - 1)
        def _next():
            gath(c0 + 2, r0, d0, gsa)
        scat(r1, d1, ssb)
        return carry
    lax.fori_loop(0, CH // 2, body, 0)
    swait(r1, d1, ssb)

    plsc.subcore_barrier()

    @pl.when(s < NS - 1)
    def _write_main():
        pltpu.sync_copy(acc.at[pl.ds(s * RPT, RPT)],
                        out_hbm.at[pl.ds(c * N + s * RPT, RPT)])

    @pl.when(s == NS - 1)
    def _write_last():
        pltpu.sync_copy(acc.at[pl.ds(15 * RPT, LAST)],
                        out_hbm.at[pl.ds(c * N + 15 * RPT, LAST)])


@functools.cache
def _sc_edge():
    return pl.kernel(
        _sc_edge_body,
        out_type=jax.ShapeDtypeStruct((NC * N, D), jnp.float32),
        mesh=plsc.VectorSubcoreMesh(core_axis_name="c", subcore_axis_name="s",
                                    num_cores=NC, num_subcores=NS),
        scratch_types=[
            pltpu.VMEM((CH, K), jnp.int32),
            pltpu.VMEM((K,), jnp.int32),
            pltpu.VMEM((K,), jnp.int32),
            pltpu.VMEM((K, D), jnp.float32),
            pltpu.VMEM((K, D), jnp.float32),
            pltpu.VMEM_SHARED((N + 128, D), jnp.float32),
            pltpu.SemaphoreType.DMA,
            pltpu.SemaphoreType.DMA,
            pltpu.SemaphoreType.DMA,
            pltpu.SemaphoreType.DMA,
        ],
    )


def _tc_first_body(x_ref, wr_ref, wro_ref, br_ref, hr_ref, hroot_ref):
    h = x_ref[...]
    hr_ref[...] = _mm_t(h, wr_ref[...])
    hroot_ref[...] = _mm_t(h, wro_ref[...]) + br_ref[...]


def _tc_mid_body(a0_ref, a1_ref, hroot_ref, wr_ref, wro_ref, br_ref,
                 hr_ref, hroot_out_ref):
    h = jax.nn.gelu(a0_ref[...] + a1_ref[...] + hroot_ref[...])
    hr_ref[...] = _mm_t(h, wr_ref[...])
    hroot_out_ref[...] = _mm_t(h, wro_ref[...]) + br_ref[...]


def _tc_final_body(a0_ref, a1_ref, hroot_ref, bidx_ref, wout_ref, bout_ref,
                   out_ref, sums_ref, cnt_ref):
    i = pl.program_id(0)

    @pl.when(i == 0)
    def _init():
        sums_ref[...] = jnp.zeros_like(sums_ref)
        cnt_ref[...] = jnp.zeros_like(cnt_ref)

    h = jax.nn.gelu(a0_ref[...] + a1_ref[...] + hroot_ref[...])
    b2 = bidx_ref[0]
    gids = lax.broadcasted_iota(jnp.int32, (G, BN), 0)
    sel = (b2 == gids).astype(jnp.float32)
    sums_ref[...] += lax.dot_general(sel, h, (((1,), (0,)), ((), ())),
                                     preferred_element_type=jnp.float32)
    cnt_ref[...] += lax.dot_general(sel, jnp.ones((BN, D), jnp.float32),
                                    (((1,), (0,)), ((), ())),
                                    preferred_element_type=jnp.float32)

    @pl.when(i == NB - 1)
    def _finish():
        pooled = sums_ref[...] / jnp.maximum(cnt_ref[...], 1.0)
        val = jnp.sum(pooled * wout_ref[...], axis=1, keepdims=True)
        out_ref[...] = val + bout_ref[0]


_row_spec = pl.BlockSpec((BN, D), lambda i: (i, 0))
_w_spec = pl.BlockSpec((D, D), lambda i: (0, 0))
_b_spec = pl.BlockSpec((1, D), lambda i: (0, 0))

_tc_first = pl.pallas_call(
    _tc_first_body,
    grid=(NB,),
    in_specs=[_row_spec, _w_spec, _w_spec, _b_spec],
    out_specs=[_row_spec, _row_spec],
    out_shape=[jax.ShapeDtypeStruct((N, D), jnp.float32)] * 2,
)

_agg0_spec = pl.BlockSpec((BN, D), lambda i: (i, 0))
_agg1_spec = pl.BlockSpec((BN, D), lambda i: (i + NB, 0))

_tc_mid = pl.pallas_call(
    _tc_mid_body,
    grid=(NB,),
    in_specs=[_agg0_spec, _agg1_spec, _row_spec, _w_spec, _w_spec, _b_spec],
    out_specs=[_row_spec, _row_spec],
    out_shape=[jax.ShapeDtypeStruct((N, D), jnp.float32)] * 2,
)

_tc_final = pl.pallas_call(
    _tc_final_body,
    grid=(NB,),
    in_specs=[
        _agg0_spec, _agg1_spec, _row_spec,
        pl.BlockSpec((1, 1, BN), lambda i: (i, 0, 0)),
        pl.BlockSpec((1, D), lambda i: (0, 0)),
        pl.BlockSpec(memory_space=pltpu.SMEM),
    ],
    out_specs=pl.BlockSpec((G, 1), lambda i: (0, 0)),
    out_shape=jax.ShapeDtypeStruct((G, 1), jnp.float32),
    scratch_shapes=[
        pltpu.VMEM((G, D), jnp.float32),
        pltpu.VMEM((G, D), jnp.float32),
    ],
)


def kernel(x, edge_index, batch_index,
           W_rel0, b_rel0, W_root0,
           W_rel1, b_rel1, W_root1,
           W_rel2, b_rel2, W_root2,
           W_rel3, b_rel3, W_root3,
           W_out, b_out):
    pad = EPAD - E
    spread = jnp.arange(pad, dtype=jnp.int32) % 128
    src = jnp.concatenate([edge_index[0], spread * 64]).reshape(NW, CH, K)
    dst = jnp.concatenate([edge_index[1], N + spread])
    bidx3 = batch_index.reshape(NB, 1, BN)

    sc_edge = _sc_edge()
    hr, hroot = _tc_first(x, W_rel0, W_root0, b_rel0.reshape(1, D))
    agg = sc_edge(hr, src, dst)
    hr, hroot = _tc_mid(agg, agg, hroot, W_rel1, W_root1, b_rel1.reshape(1, D))
    agg = sc_edge(hr, src, dst)
    hr, hroot = _tc_mid(agg, agg, hroot, W_rel2, W_root2, b_rel2.reshape(1, D))
    agg = sc_edge(hr, src, dst)
    hr, hroot = _tc_mid(agg, agg, hroot, W_rel3, W_root3, b_rel3.reshape(1, D))
    agg = sc_edge(hr, src, dst)
    out = _tc_final(agg, agg, hroot, bidx3, W_out, b_out)
    return out


# speedup vs baseline: 1.0442x; 1.0442x over previous
import functools

import jax
import jax.numpy as jnp
from jax import lax
from jax.experimental import pallas as pl
from jax.experimental.pallas import tpu as pltpu
from jax.experimental.pallas import tpu_sc as plsc

N = 10000
E = 320000
D = 128
G = 256

NC = 2
NS = 16
NW = NC * NS

K = 128
CH = 80
EPW = CH * K
EPAD = NW * EPW
RPT = 624
LAST = N - 15 * RPT

BN = 2000
NB = N // BN


def _mm_t(a, b):
    return lax.dot_general(a, b, (((1,), (1,)), ((), ())),
                           preferred_element_type=jnp.float32)


def _sc_edge_body(hr_hbm, src_hbm, dst_hbm, out_hbm,
                  src_v, d0, d1, r0, r1, acc,
                  gsa, gsb, ssa, ssb):
    c = lax.axis_index("c")
    s = lax.axis_index("s")
    wid = c * NS + s
    base = wid * EPW

    pltpu.sync_copy(src_hbm.at[wid], src_v)

    def zrow(i, carry):
        def zcol(j, carry2):
            r0[i, pl.ds(j * 16, 16)] = jnp.zeros((16,), jnp.float32)
            return carry2
        return lax.fori_loop(0, D // 16, zcol, carry)
    lax.fori_loop(0, K, zrow, 0)

    def zslab(i, carry):
        pltpu.sync_copy(r0, acc.at[pl.ds(s * RPT + i * K, K)])
        return carry
    lax.fori_loop(0, RPT // K, zslab, 0)

    @pl.when(s < NS - 1)
    def _zero_rem():
        pltpu.sync_copy(r0.at[pl.ds(0, RPT - (RPT // K) * K)],
                        acc.at[pl.ds(s * RPT + (RPT // K) * K,
                                     RPT - (RPT // K) * K)])

    @pl.when(s == NS - 1)
    def _zero_tail():
        pltpu.sync_copy(r0, acc.at[pl.ds(15 * RPT + (RPT // K) * K, K)])
    plsc.subcore_barrier()

    def gath(i, buf, dbuf, sem):
        pltpu.async_copy(hr_hbm.at[src_v.at[i]], buf, sem)
        pltpu.async_copy(dst_hbm.at[pl.ds(base + i * K, K)], dbuf, sem)

    def gwait(i, buf, dbuf, sem):
        pltpu.make_async_copy(hr_hbm.at[src_v.at[i]], buf, sem).wait()
        pltpu.make_async_copy(dst_hbm.at[pl.ds(base + i * K, K)], dbuf,
                              sem).wait()

    def scat(buf, dbuf, sem):
        pltpu.async_copy(buf, acc.at[dbuf], sem, add=True)

    def swait(buf, dbuf, sem):
        pltpu.make_async_copy(buf, acc.at[dbuf], sem).wait()

    gath(0, r0, d0, gsa)

    def body(t, carry):
        c0 = 2 * t
        gwait(c0, r0, d0, gsa)

        @pl.when(t > 0)
        def _drain_prev():
            swait(r1, d1, ssb)
        gath(c0 + 1, r1, d1, gsb)
        scat(r0, d0, ssa)
        gwait(c0 + 1, r1, d1, gsb)
        swait(r0, d0, ssa)

        @pl.when(t < CH // 2 - 1)
        def _next():
            gath(c0 + 2, r0, d0, gsa)
        scat(r1, d1, ssb)
        return carry
    lax.fori_loop(0, CH // 2, body, 0)
    swait(r1, d1, ssb)

    plsc.subcore_barrier()

    @pl.when(s < NS - 1)
    def _write_main():
        pltpu.sync_copy(acc.at[pl.ds(s * RPT, RPT)],
                        out_hbm.at[pl.ds(c * N + s * RPT, RPT)])

    @pl.when(s == NS - 1)
    def _write_last():
        pltpu.sync_copy(acc.at[pl.ds(15 * RPT, LAST)],
                        out_hbm.at[pl.ds(c * N + 15 * RPT, LAST)])


@functools.cache
def _sc_edge():
    return pl.kernel(
        _sc_edge_body,
        out_type=jax.ShapeDtypeStruct((NC * N, D), jnp.float32),
        mesh=plsc.VectorSubcoreMesh(core_axis_name="c", subcore_axis_name="s",
                                    num_cores=NC, num_subcores=NS),
        scratch_types=[
            pltpu.VMEM((CH, K), jnp.int32),
            pltpu.VMEM((K,), jnp.int32),
            pltpu.VMEM((K,), jnp.int32),
            pltpu.VMEM((K, D), jnp.float32),
            pltpu.VMEM((K, D), jnp.float32),
            pltpu.VMEM_SHARED((N + 128, D), jnp.float32),
            pltpu.SemaphoreType.DMA,
            pltpu.SemaphoreType.DMA,
            pltpu.SemaphoreType.DMA,
            pltpu.SemaphoreType.DMA,
        ],
    )


def _tc_layer_body(a0_ref, a1_ref, h_ref, wr_ref, wro_ref, br_ref, out_ref):
    agg = a0_ref[...] + a1_ref[...]
    out_ref[...] = jax.nn.gelu(_mm_t(agg, wr_ref[...]) + br_ref[...]
                               + _mm_t(h_ref[...], wro_ref[...]))


def _tc_final_body(a0_ref, a1_ref, h_ref, wr_ref, wro_ref, br_ref,
                   bidx_ref, wout_ref, bout_ref,
                   out_ref, sums_ref, cnt_ref):
    i = pl.program_id(0)

    @pl.when(i == 0)
    def _init():
        sums_ref[...] = jnp.zeros_like(sums_ref)
        cnt_ref[...] = jnp.zeros_like(cnt_ref)

    agg = a0_ref[...] + a1_ref[...]
    h = jax.nn.gelu(_mm_t(agg, wr_ref[...]) + br_ref[...]
                    + _mm_t(h_ref[...], wro_ref[...]))
    b2 = bidx_ref[0]
    gids = lax.broadcasted_iota(jnp.int32, (G, BN), 0)
    sel = (b2 == gids).astype(jnp.float32)
    sums_ref[...] += lax.dot_general(sel, h, (((1,), (0,)), ((), ())),
                                     preferred_element_type=jnp.float32)
    cnt_ref[...] += lax.dot_general(sel, jnp.ones((BN, D), jnp.float32),
                                    (((1,), (0,)), ((), ())),
                                    preferred_element_type=jnp.float32)

    @pl.when(i == NB - 1)
    def _finish():
        pooled = sums_ref[...] / jnp.maximum(cnt_ref[...], 1.0)
        val = jnp.sum(pooled * wout_ref[...], axis=1, keepdims=True)
        out_ref[...] = val + bout_ref[0]


_row_spec = pl.BlockSpec((BN, D), lambda i: (i, 0))
_w_spec = pl.BlockSpec((D, D), lambda i: (0, 0))
_b_spec = pl.BlockSpec((1, D), lambda i: (0, 0))

_agg0_spec = pl.BlockSpec((BN, D), lambda i: (i, 0))
_agg1_spec = pl.BlockSpec((BN, D), lambda i: (i + NB, 0))

_tc_layer = pl.pallas_call(
    _tc_layer_body,
    grid=(NB,),
    in_specs=[_agg0_spec, _agg1_spec, _row_spec, _w_spec, _w_spec, _b_spec],
    out_specs=_row_spec,
    out_shape=jax.ShapeDtypeStruct((N, D), jnp.float32),
)

_tc_final = pl.pallas_call(
    _tc_final_body,
    grid=(NB,),
    in_specs=[
        _agg0_spec, _agg1_spec, _row_spec, _w_spec, _w_spec, _b_spec,
        pl.BlockSpec((1, 1, BN), lambda i: (i, 0, 0)),
        pl.BlockSpec((1, D), lambda i: (0, 0)),
        pl.BlockSpec(memory_space=pltpu.SMEM),
    ],
    out_specs=pl.BlockSpec((G, 1), lambda i: (0, 0)),
    out_shape=jax.ShapeDtypeStruct((G, 1), jnp.float32),
    scratch_shapes=[
        pltpu.VMEM((G, D), jnp.float32),
        pltpu.VMEM((G, D), jnp.float32),
    ],
)


def kernel(x, edge_index, batch_index,
           W_rel0, b_rel0, W_root0,
           W_rel1, b_rel1, W_root1,
           W_rel2, b_rel2, W_root2,
           W_rel3, b_rel3, W_root3,
           W_out, b_out):
    pad = EPAD - E
    spread = jnp.arange(pad, dtype=jnp.int32) % 128
    src = jnp.concatenate([edge_index[0], spread * 64]).reshape(NW, CH, K)
    dst = jnp.concatenate([edge_index[1], N + spread])
    bidx3 = batch_index.reshape(NB, 1, BN)

    sc_edge = _sc_edge()
    agg = sc_edge(x, src, dst)
    h = _tc_layer(agg, agg, x, W_rel0, W_root0, b_rel0.reshape(1, D))
    agg = sc_edge(h, src, dst)
    h = _tc_layer(agg, agg, h, W_rel1, W_root1, b_rel1.reshape(1, D))
    agg = sc_edge(h, src, dst)
    h = _tc_layer(agg, agg, h, W_rel2, W_root2, b_rel2.reshape(1, D))
    agg = sc_edge(h, src, dst)
    out = _tc_final(agg, agg, h, W_rel3, W_root3, b_rel3.reshape(1, D),
                    bidx3, W_out, b_out)
    return out
